# baseline (speedup 1.0000x reference)
import functools

import jax
import jax.numpy as jnp
from jax.experimental import pallas as pl
from jax.experimental.pallas import tpu as pltpu


def _router_body(x_ref, w0_ref, b0_ref, w1_ref, b1_ref, w2_ref, b2_ref,
                 out_ref, lg_ref, *, s_t: int, tpb: int, n_tiles: int):
    i = pl.program_id(0)
    s = i % tpb
    par = (i // tpb) % 2

    @pl.when(i < n_tiles)
    def _logits():
        h = jnp.dot(x_ref[...], w0_ref[...],
                    preferred_element_type=jnp.float32)
        h = jnp.maximum(h + b0_ref[...], 0.0)
        h = jnp.dot(h, w1_ref[...], preferred_element_type=jnp.float32)
        h = jnp.maximum(h + b1_ref[...], 0.0)
        logits = jnp.dot(h, w2_ref[...], preferred_element_type=jnp.float32)
        lg_ref[par, pl.ds(s * s_t, s_t), :] = logits + b2_ref[...]

    @pl.when((i >= tpb) & (s == 0))
    def _softmax_prev_batch():
        lg = lg_ref[1 - par]
        m = jnp.max(lg, axis=0, keepdims=True)
        e = jnp.exp(lg - m)
        out_ref[0] = e / jnp.sum(e, axis=0, keepdims=True)


@jax.jit
def kernel(x, W0, b0, W1, b1, W2, b2):
    B, S, D = x.shape
    H0 = W0.shape[1]
    H1 = W1.shape[1]
    E = W2.shape[1]
    S_T = 2048
    tpb = S // S_T
    n_tiles = B * S // S_T

    x2 = x.reshape(B * S, D)
    b0r = b0.reshape(1, H0)
    b1r = b1.reshape(1, H1)
    b2r = b2.reshape(1, E)

    body = functools.partial(_router_body, s_t=S_T, tpb=tpb, n_tiles=n_tiles)
    return pl.pallas_call(
        body,
        grid=(n_tiles + 1,),
        in_specs=[
            pl.BlockSpec((S_T, D), lambda i: (jnp.minimum(i, n_tiles - 1), 0)),
            pl.BlockSpec((D, H0), lambda i: (0, 0)),
            pl.BlockSpec((1, H0), lambda i: (0, 0)),
            pl.BlockSpec((H0, H1), lambda i: (0, 0)),
            pl.BlockSpec((1, H1), lambda i: (0, 0)),
            pl.BlockSpec((H1, E), lambda i: (0, 0)),
            pl.BlockSpec((1, E), lambda i: (0, 0)),
        ],
        out_specs=pl.BlockSpec(
            (1, S, E), lambda i: (jnp.maximum(i // tpb - 1, 0), 0, 0)
        ),
        out_shape=jax.ShapeDtypeStruct((B, S, E), jnp.float32),
        scratch_shapes=[pltpu.VMEM((2, S, E), jnp.float32)],
        compiler_params=pltpu.CompilerParams(
            dimension_semantics=("arbitrary",),
            vmem_limit_bytes=100 * 1024 * 1024,
        ),
    )(x2, W0, b0r, W1, b1r, W2, b2r)
